# baseline (speedup 1.0000x reference)
import functools

import jax
import jax.numpy as jnp
from jax import lax
from jax.experimental import pallas as pl
from jax.experimental.pallas import tpu as pltpu
from jax.experimental.pallas import tpu_sc as plsc

NC = 2
NS = 16
NW = NC * NS
CH = 128


def _gather_call(n, d, n_per_w, n_ch):
    mesh = plsc.VectorSubcoreMesh(core_axis_name="c", subcore_axis_name="s")

    @functools.partial(
        pl.kernel,
        mesh=mesh,
        out_type=jax.ShapeDtypeStruct((n, d), jnp.float32),
        scratch_types=[
            pltpu.VMEM((n_ch, CH), jnp.int32),
            pltpu.VMEM((CH, d), jnp.float32),
            pltpu.SemaphoreType.DMA,
        ],
        compiler_params=pltpu.CompilerParams(use_tc_tiling_on_sc=False),
    )
    def k(idx_hbm, table_hbm, out_hbm, idx_v, rows_v, sem):
        wid = lax.axis_index("s") * NC + lax.axis_index("c")
        base = wid * n_per_w
        pltpu.sync_copy(idx_hbm.at[wid], idx_v)

        def body(j, carry):
            pltpu.async_copy(table_hbm.at[idx_v.at[j]], rows_v, sem).wait()
            pltpu.sync_copy(rows_v, out_hbm.at[pl.ds(base + j * CH, CH)])
            return carry

        lax.fori_loop(0, n_ch, body, 0)

    return k


def kernel(token_ids, emb):
    b, s = token_ids.shape
    n = b * s
    d = emb.shape[1]
    n_per_w = n // NW
    n_ch = n_per_w // CH
    idx3 = token_ids.reshape(NW, n_ch, CH)
    out = _gather_call(n, d, n_per_w, n_ch)(idx3, emb)
    return out.reshape(b, s, d)


# speedup vs baseline: 1.0899x; 1.0899x over previous
import functools

import jax
import jax.numpy as jnp
from jax import lax
from jax.experimental import pallas as pl
from jax.experimental.pallas import tpu as pltpu
from jax.experimental.pallas import tpu_sc as plsc

NC = 2
NS = 16
NW = NC * NS
CH = 512


def _gather_call(n, d, n_per_w, n_ch):
    mesh = plsc.VectorSubcoreMesh(core_axis_name="c", subcore_axis_name="s")

    @functools.partial(
        pl.kernel,
        mesh=mesh,
        out_type=jax.ShapeDtypeStruct((n, d), jnp.float32),
        scratch_types=[
            pltpu.VMEM((n_ch, CH), jnp.int32),
            pltpu.VMEM((CH, d), jnp.float32),
            pltpu.SemaphoreType.DMA,
        ],
        compiler_params=pltpu.CompilerParams(use_tc_tiling_on_sc=False),
    )
    def k(idx_hbm, table_hbm, out_hbm, idx_v, rows_v, sem):
        wid = lax.axis_index("s") * NC + lax.axis_index("c")
        base = wid * n_per_w
        pltpu.sync_copy(idx_hbm.at[wid], idx_v)

        def body(j, carry):
            pltpu.async_copy(table_hbm.at[idx_v.at[j]], rows_v, sem).wait()
            pltpu.sync_copy(rows_v, out_hbm.at[pl.ds(base + j * CH, CH)])
            return carry

        lax.fori_loop(0, n_ch, body, 0)

    return k


def kernel(token_ids, emb):
    b, s = token_ids.shape
    n = b * s
    d = emb.shape[1]
    n_per_w = n // NW
    n_ch = n_per_w // CH
    idx3 = token_ids.reshape(NW, n_ch, CH)
    out = _gather_call(n, d, n_per_w, n_ch)(idx3, emb)
    return out.reshape(b, s, d)


# speedup vs baseline: 1.1147x; 1.0228x over previous
import functools

import jax
import jax.numpy as jnp
from jax import lax
from jax.experimental import pallas as pl
from jax.experimental.pallas import tpu as pltpu
from jax.experimental.pallas import tpu_sc as plsc

NC = 2
NS = 16
NW = NC * NS
CH = 256
R = 4


def _gather_call(n, d, n_per_w, n_ch):
    mesh = plsc.VectorSubcoreMesh(core_axis_name="c", subcore_axis_name="s")

    @functools.partial(
        pl.kernel,
        mesh=mesh,
        out_type=jax.ShapeDtypeStruct((n, d), jnp.float32),
        scratch_types=[
            pltpu.VMEM((n_ch, CH), jnp.int32),
            pltpu.VMEM((R, CH, d), jnp.float32),
            pltpu.SemaphoreType.DMA((R,)),
            pltpu.SemaphoreType.DMA((R,)),
        ],
        compiler_params=pltpu.CompilerParams(use_tc_tiling_on_sc=False),
    )
    def k(idx_hbm, table_hbm, out_hbm, idx_v, rows, gsem, wsem):
        wid = lax.axis_index("s") * NC + lax.axis_index("c")
        base = wid * n_per_w
        pltpu.sync_copy(idx_hbm.at[wid], idx_v)

        def start_g(t, b):
            pltpu.async_copy(table_hbm.at[idx_v.at[t]], rows.at[b], gsem.at[b])

        def wait_g(b):
            pltpu.make_async_copy(
                table_hbm.at[idx_v.at[0]], rows.at[b], gsem.at[b]
            ).wait()

        def start_w(t, b):
            pltpu.async_copy(
                rows.at[b], out_hbm.at[pl.ds(base + t * CH, CH)], wsem.at[b]
            )

        def wait_w(b):
            pltpu.make_async_copy(
                rows.at[b], out_hbm.at[pl.ds(base, CH)], wsem.at[b]
            ).wait()

        start_g(0, 0)
        start_g(1, 1)
        start_g(2, 2)
        wait_g(0)
        start_w(0, 0)
        start_g(3, 3)
        wait_g(1)
        start_w(1, 1)

        @pl.loop(0, (n_ch - 4) // R)
        def body(g):
            t0 = 2 + R * g
            for i in range(R):
                wait_w(i)
                start_g(t0 + 2 + i, i)
                s = (2 + i) % R
                wait_g(s)
                start_w(t0 + i, s)

        wait_g(2)
        start_w(n_ch - 2, 2)
        wait_g(3)
        start_w(n_ch - 1, 3)
        for i in range(R):
            wait_w(i)

    return k


def kernel(token_ids, emb):
    b, s = token_ids.shape
    n = b * s
    d = emb.shape[1]
    n_per_w = n // NW
    n_ch = n_per_w // CH
    idx3 = token_ids.reshape(NW, n_ch, CH)
    out = _gather_call(n, d, n_per_w, n_ch)(idx3, emb)
    return out.reshape(b, s, d)


# speedup vs baseline: 1.1155x; 1.0007x over previous
import functools

import jax
import jax.numpy as jnp
from jax import lax
from jax.experimental import pallas as pl
from jax.experimental.pallas import tpu as pltpu
from jax.experimental.pallas import tpu_sc as plsc

NC = 2
NS = 16
NW = NC * NS
R = 4


def _gather_call(b, s, d):
    bw = b // NW
    mesh = plsc.VectorSubcoreMesh(core_axis_name="c", subcore_axis_name="s")

    @functools.partial(
        pl.kernel,
        mesh=mesh,
        out_type=jax.ShapeDtypeStruct((b, s, d), jnp.float32),
        scratch_types=[
            pltpu.VMEM((s, bw), jnp.int32),
            pltpu.VMEM((R, bw, d), jnp.float32),
            pltpu.SemaphoreType.DMA((R,)),
            pltpu.SemaphoreType.DMA((R,)),
        ],
        compiler_params=pltpu.CompilerParams(use_tc_tiling_on_sc=False),
    )
    def k(idx_hbm, table_hbm, out_hbm, idx_v, rows, gsem, wsem):
        wid = lax.axis_index("s") * NC + lax.axis_index("c")
        b0 = wid * bw
        pltpu.sync_copy(idx_hbm.at[:, pl.ds(b0, bw)], idx_v)

        def start_g(t, buf):
            pltpu.async_copy(table_hbm.at[idx_v.at[t]], rows.at[buf], gsem.at[buf])

        def wait_g(buf):
            pltpu.make_async_copy(
                table_hbm.at[idx_v.at[0]], rows.at[buf], gsem.at[buf]
            ).wait()

        def start_w(t, buf):
            pltpu.async_copy(
                rows.at[buf], out_hbm.at[pl.ds(b0, bw), t], wsem.at[buf]
            )

        def wait_w(buf):
            pltpu.make_async_copy(
                rows.at[buf], out_hbm.at[pl.ds(b0, bw), 0], wsem.at[buf]
            ).wait()

        start_g(0, 0)
        start_g(1, 1)
        start_g(2, 2)
        wait_g(0)
        start_w(0, 0)
        start_g(3, 3)
        wait_g(1)
        start_w(1, 1)

        @pl.loop(0, (s - 4) // R)
        def body(g):
            t0 = 2 + R * g
            for i in range(R):
                wait_w(i)
                start_g(t0 + 2 + i, i)
                sl = (2 + i) % R
                wait_g(sl)
                start_w(t0 + i, sl)

        wait_g(2)
        start_w(s - 2, 2)
        wait_g(3)
        start_w(s - 1, 3)
        for i in range(R):
            wait_w(i)

    return k


def kernel(token_ids, emb):
    b, s = token_ids.shape
    d = emb.shape[1]
    return _gather_call(b, s, d)(token_ids.T, emb)
